# Initial kernel scaffold; baseline (speedup 1.0000x reference)
import functools

import jax
import jax.numpy as jnp
from jax import lax
from jax.experimental import pallas as pl
from jax.experimental.pallas import tpu as pltpu
from jax.experimental.pallas import tpu_sc as plsc

EMBED_DIM = 32
NUM_CORES = 2
NUM_SUBCORES = 16
NUM_WORKERS = NUM_CORES * NUM_SUBCORES
CHUNK = 1024


def _make_gather(total_b):
    assert total_b % (NUM_WORKERS * CHUNK) == 0
    b_per_w = total_b // NUM_WORKERS
    n_chunks = b_per_w // CHUNK
    mesh = plsc.VectorSubcoreMesh(
        core_axis_name="c", subcore_axis_name="s",
        num_cores=NUM_CORES, num_subcores=NUM_SUBCORES)

    @functools.partial(
        pl.kernel,
        mesh=mesh,
        out_type=jax.ShapeDtypeStruct((total_b, EMBED_DIM), jnp.float32),
        scratch_types=[
            pltpu.VMEM((CHUNK,), jnp.int32),
            pltpu.VMEM((CHUNK, EMBED_DIM), jnp.float32),
            pltpu.SemaphoreType.DMA,
        ],
    )
    def gather_kernel(ids_hbm, table_hbm, out_hbm, idx_v, rows_v, sem):
        wid = lax.axis_index("s") * NUM_CORES + lax.axis_index("c")
        base = wid * b_per_w

        def body(i, carry):
            off = base + i * CHUNK
            pltpu.sync_copy(ids_hbm.at[pl.ds(off, CHUNK)], idx_v)
            pltpu.async_copy(table_hbm.at[idx_v], rows_v, sem).wait()
            pltpu.sync_copy(rows_v, out_hbm.at[pl.ds(off, CHUNK)])
            return carry

        lax.fori_loop(0, n_chunks, body, 0)

    return gather_kernel


def kernel(inputs, table):
    flat_ids = inputs.reshape(-1).astype(jnp.int32)
    out_flat = _make_gather(flat_ids.shape[0])(flat_ids, table)
    return out_flat.reshape(inputs.shape + (EMBED_DIM,))


# speedup vs baseline: 6.2011x; 6.2011x over previous
import functools

import jax
import jax.numpy as jnp
from jax import lax
from jax.experimental import pallas as pl
from jax.experimental.pallas import tpu as pltpu
from jax.experimental.pallas import tpu_sc as plsc

EMBED_DIM = 32
NUM_CORES = 2
NUM_SUBCORES = 16
NUM_WORKERS = NUM_CORES * NUM_SUBCORES
CHUNK = 1024


def _make_gather(total_b):
    assert total_b % (NUM_WORKERS * CHUNK) == 0
    b_per_w = total_b // NUM_WORKERS
    n_chunks = b_per_w // CHUNK
    mesh = plsc.VectorSubcoreMesh(
        core_axis_name="c", subcore_axis_name="s",
        num_cores=NUM_CORES, num_subcores=NUM_SUBCORES)

    @functools.partial(
        pl.kernel,
        mesh=mesh,
        compiler_params=pltpu.CompilerParams(use_tc_tiling_on_sc=False),
        out_type=jax.ShapeDtypeStruct((total_b, EMBED_DIM), jnp.float32),
        scratch_types=[
            pltpu.VMEM((CHUNK,), jnp.int32),
            pltpu.VMEM((CHUNK, EMBED_DIM), jnp.float32),
            pltpu.SemaphoreType.DMA,
        ],
    )
    def gather_kernel(ids_hbm, table_hbm, out_hbm, idx_v, rows_v, sem):
        wid = lax.axis_index("s") * NUM_CORES + lax.axis_index("c")
        base = wid * b_per_w

        def body(i, carry):
            off = base + i * CHUNK
            pltpu.sync_copy(ids_hbm.at[pl.ds(off, CHUNK)], idx_v)
            pltpu.async_copy(table_hbm.at[idx_v], rows_v, sem).wait()
            pltpu.sync_copy(rows_v, out_hbm.at[pl.ds(off, CHUNK)])
            return carry

        lax.fori_loop(0, n_chunks, body, 0)

    return gather_kernel


def kernel(inputs, table):
    flat_ids = inputs.reshape(-1).astype(jnp.int32)
    out_flat = _make_gather(flat_ids.shape[0])(flat_ids, table)
    return out_flat.reshape(inputs.shape + (EMBED_DIM,))


# speedup vs baseline: 6.3072x; 1.0171x over previous
import functools

import jax
import jax.numpy as jnp
from jax import lax
from jax.experimental import pallas as pl
from jax.experimental.pallas import tpu as pltpu
from jax.experimental.pallas import tpu_sc as plsc

EMBED_DIM = 32
NUM_CORES = 2
NUM_SUBCORES = 16
NUM_WORKERS = NUM_CORES * NUM_SUBCORES
CHUNK = 1664


def _make_gather(total_b):
    assert total_b % (NUM_WORKERS * CHUNK) == 0
    b_per_w = total_b // NUM_WORKERS
    n_chunks = b_per_w // CHUNK
    mesh = plsc.VectorSubcoreMesh(
        core_axis_name="c", subcore_axis_name="s",
        num_cores=NUM_CORES, num_subcores=NUM_SUBCORES)

    @functools.partial(
        pl.kernel,
        mesh=mesh,
        compiler_params=pltpu.CompilerParams(use_tc_tiling_on_sc=False),
        out_type=jax.ShapeDtypeStruct((total_b, EMBED_DIM), jnp.float32),
        scratch_types=[
            pltpu.VMEM((n_chunks, CHUNK), jnp.int32),
            pltpu.VMEM((2, CHUNK, EMBED_DIM), jnp.float32),
            pltpu.SemaphoreType.DMA,
            pltpu.SemaphoreType.DMA,
            pltpu.SemaphoreType.DMA,
        ],
    )
    def gather_kernel(ids_hbm, table_hbm, out_hbm, idx_v, rows_v,
                      sem_g0, sem_g1, sem_o):
        wid = lax.axis_index("s") * NUM_CORES + lax.axis_index("c")
        base = wid * b_per_w
        sems = (sem_g0, sem_g1)

        pltpu.sync_copy(ids_hbm.at[wid], idx_v)

        def gather_start(i):
            s = i % 2
            pltpu.make_async_copy(
                table_hbm.at[idx_v.at[i]], rows_v.at[s], sems[s]).start()

        def gather_wait(i):
            s = i % 2
            pltpu.make_async_copy(
                table_hbm.at[idx_v.at[i]], rows_v.at[s], sems[s]).wait()

        def out_copy(i):
            s = i % 2
            return pltpu.make_async_copy(
                rows_v.at[s], out_hbm.at[pl.ds(base + i * CHUNK, CHUNK)],
                sem_o)

        gather_start(0)
        for i in range(n_chunks):
            if i + 1 < n_chunks:
                if i >= 1:
                    out_copy(i - 1).wait()
                gather_start(i + 1)
            gather_wait(i)
            out_copy(i).start()
        out_copy(n_chunks - 2).wait()
        out_copy(n_chunks - 1).wait()

    return gather_kernel


def kernel(inputs, table):
    flat_ids = inputs.reshape(-1).astype(jnp.int32)
    total_b = flat_ids.shape[0]
    ids3d = flat_ids.reshape(NUM_WORKERS, total_b // (NUM_WORKERS * CHUNK),
                             CHUNK)
    out_flat = _make_gather(total_b)(ids3d, table)
    return out_flat.reshape(inputs.shape + (EMBED_DIM,))


# speedup vs baseline: 6.3096x; 1.0004x over previous
import functools

import jax
import jax.numpy as jnp
from jax import lax
from jax.experimental import pallas as pl
from jax.experimental.pallas import tpu as pltpu
from jax.experimental.pallas import tpu_sc as plsc

EMBED_DIM = 32
NUM_CORES = 2
NUM_SUBCORES = 16
NUM_WORKERS = NUM_CORES * NUM_SUBCORES
CHUNK = 1664


def _make_gather(total_b, vocab):
    assert total_b % (NUM_WORKERS * CHUNK) == 0
    b_per_w = total_b // NUM_WORKERS
    n_chunks = b_per_w // CHUNK
    mesh = plsc.VectorSubcoreMesh(
        core_axis_name="c", subcore_axis_name="s",
        num_cores=NUM_CORES, num_subcores=NUM_SUBCORES)

    @functools.partial(
        pl.kernel,
        mesh=mesh,
        compiler_params=pltpu.CompilerParams(use_tc_tiling_on_sc=False),
        out_type=jax.ShapeDtypeStruct((total_b, EMBED_DIM), jnp.float32),
        scratch_types=[
            pltpu.VMEM((b_per_w,), jnp.int32),
            pltpu.VMEM((2, CHUNK, EMBED_DIM), jnp.float32),
            pltpu.SemaphoreType.DMA,
            pltpu.SemaphoreType.DMA,
            pltpu.SemaphoreType.DMA,
        ],
    )
    def gather_kernel(ids_hbm, table_hbm, out_hbm, idx_v, rows_v,
                      sem_g0, sem_g1, sem_o):
        wid = lax.axis_index("s") * NUM_CORES + lax.axis_index("c")
        base = wid * b_per_w
        sems = (sem_g0, sem_g1)
        table2 = table_hbm
        out2 = out_hbm

        pltpu.sync_copy(ids_hbm.at[wid], idx_v)

        def gather_copy(i):
            s = i % 2
            return pltpu.make_async_copy(
                table2.at[idx_v.at[pl.ds(i * CHUNK, CHUNK)]],
                rows_v.at[s], sems[s])

        def out_copy(i):
            s = i % 2
            return pltpu.make_async_copy(
                rows_v.at[s], out2.at[pl.ds(base + i * CHUNK, CHUNK)],
                sem_o)

        gather_copy(0).start()
        for i in range(n_chunks):
            if i + 1 < n_chunks:
                if i >= 1:
                    out_copy(i - 1).wait()
                gather_copy(i + 1).start()
            gather_copy(i).wait()
            out_copy(i).start()
        out_copy(n_chunks - 2).wait()
        out_copy(n_chunks - 1).wait()

    return gather_kernel


def kernel(inputs, table):
    flat_ids = inputs.reshape(-1).astype(jnp.int32)
    total_b = flat_ids.shape[0]
    ids2 = flat_ids.reshape(NUM_WORKERS, total_b // NUM_WORKERS)
    out = _make_gather(total_b, table.shape[0])(ids2, table)
    return out.reshape(inputs.shape + (EMBED_DIM,))


# speedup vs baseline: 6.3098x; 1.0000x over previous
import functools

import jax
import jax.numpy as jnp
from jax import lax
from jax.experimental import pallas as pl
from jax.experimental.pallas import tpu as pltpu
from jax.experimental.pallas import tpu_sc as plsc

EMBED_DIM = 32
NUM_CORES = 2
NUM_SUBCORES = 16
NUM_WORKERS = NUM_CORES * NUM_SUBCORES
CHUNK = 1664


def _make_gather(total_b):
    assert total_b % (NUM_WORKERS * CHUNK) == 0
    b_per_w = total_b // NUM_WORKERS
    n_chunks = b_per_w // CHUNK
    mesh = plsc.VectorSubcoreMesh(
        core_axis_name="c", subcore_axis_name="s",
        num_cores=NUM_CORES, num_subcores=NUM_SUBCORES)

    @functools.partial(
        pl.kernel,
        mesh=mesh,
        compiler_params=pltpu.CompilerParams(use_tc_tiling_on_sc=False),
        out_type=jax.ShapeDtypeStruct((total_b, EMBED_DIM), jnp.float32),
        scratch_types=[
            pltpu.VMEM((b_per_w,), jnp.int32),
            pltpu.VMEM((2, CHUNK, EMBED_DIM), jnp.float32),
            pltpu.SemaphoreType.DMA,
            pltpu.SemaphoreType.DMA,
            pltpu.SemaphoreType.DMA,
        ],
    )
    def gather_kernel(ids_hbm, table_hbm, out_hbm, idx_v, rows_v,
                      sem_g0, sem_g1, sem_o):
        wid = lax.axis_index("s") * NUM_CORES + lax.axis_index("c")
        base = wid * b_per_w
        sems = (sem_g0, sem_g1)

        pltpu.sync_copy(ids_hbm.at[wid], idx_v)

        def gather_copy(i):
            s = i % 2
            return pltpu.make_async_copy(
                table_hbm.at[idx_v.at[pl.ds(i * CHUNK, CHUNK)]],
                rows_v.at[s], sems[s])

        def out_copy(i):
            s = i % 2
            return pltpu.make_async_copy(
                rows_v.at[s],
                out_hbm.at[pl.ds(base + i * CHUNK, CHUNK)],
                sem_o)

        gather_copy(0).start()
        for i in range(n_chunks):
            if i + 1 < n_chunks:
                if i >= 1:
                    out_copy(i - 1).wait()
                gather_copy(i + 1).start()
            gather_copy(i).wait()
            out_copy(i).start()
        out_copy(n_chunks - 2).wait()
        out_copy(n_chunks - 1).wait()

    return gather_kernel


def kernel(inputs, table):
    flat_ids = inputs.reshape(-1).astype(jnp.int32)
    total_b = flat_ids.shape[0]
    ids2 = flat_ids.reshape(NUM_WORKERS, total_b // NUM_WORKERS)
    flat_out = _make_gather(total_b)(ids2, table)
    return flat_out.reshape(inputs.shape + (EMBED_DIM,))


# speedup vs baseline: 6.3147x; 1.0008x over previous
import functools

import jax
import jax.numpy as jnp
from jax import lax
from jax.experimental import pallas as pl
from jax.experimental.pallas import tpu as pltpu
from jax.experimental.pallas import tpu_sc as plsc

EMBED_DIM = 32
NUM_CORES = 2
NUM_SUBCORES = 16
NUM_WORKERS = NUM_CORES * NUM_SUBCORES
CHUNK = 832
NUM_SLOTS = 4


def _make_gather(total_b):
    assert total_b % (NUM_WORKERS * CHUNK) == 0
    b_per_w = total_b // NUM_WORKERS
    n_chunks = b_per_w // CHUNK
    assert n_chunks >= NUM_SLOTS
    mesh = plsc.VectorSubcoreMesh(
        core_axis_name="c", subcore_axis_name="s",
        num_cores=NUM_CORES, num_subcores=NUM_SUBCORES)

    @functools.partial(
        pl.kernel,
        mesh=mesh,
        compiler_params=pltpu.CompilerParams(use_tc_tiling_on_sc=False),
        out_type=jax.ShapeDtypeStruct((total_b, EMBED_DIM), jnp.float32),
        scratch_types=[
            pltpu.VMEM((b_per_w,), jnp.int32),
            pltpu.VMEM((NUM_SLOTS, CHUNK, EMBED_DIM), jnp.float32),
            pltpu.SemaphoreType.DMA,
            pltpu.SemaphoreType.DMA,
            pltpu.SemaphoreType.DMA,
            pltpu.SemaphoreType.DMA,
            pltpu.SemaphoreType.DMA,
        ],
    )
    def gather_kernel(ids_hbm, table_hbm, out_hbm, idx_v, rows_v,
                      sem_g0, sem_g1, sem_g2, sem_g3, sem_o):
        wid = lax.axis_index("s") * NUM_CORES + lax.axis_index("c")
        base = wid * b_per_w
        sems = (sem_g0, sem_g1, sem_g2, sem_g3)

        pltpu.sync_copy(ids_hbm.at[wid], idx_v)

        def gather_copy(i):
            s = i % NUM_SLOTS
            return pltpu.make_async_copy(
                table_hbm.at[idx_v.at[pl.ds(i * CHUNK, CHUNK)]],
                rows_v.at[s], sems[s])

        def out_copy(i):
            s = i % NUM_SLOTS
            return pltpu.make_async_copy(
                rows_v.at[s],
                out_hbm.at[pl.ds(base + i * CHUNK, CHUNK)],
                sem_o)

        for i in range(NUM_SLOTS - 1):
            gather_copy(i).start()
        for i in range(n_chunks):
            gather_copy(i).wait()
            out_copy(i).start()
            j = i + NUM_SLOTS - 1
            if j < n_chunks:
                if i >= 1:
                    out_copy(i - 1).wait()
                gather_copy(j).start()
        for k in range(max(0, n_chunks - NUM_SLOTS), n_chunks):
            out_copy(k).wait()

    return gather_kernel


def kernel(inputs, table):
    flat_ids = inputs.reshape(-1).astype(jnp.int32)
    total_b = flat_ids.shape[0]
    ids2 = flat_ids.reshape(NUM_WORKERS, total_b // NUM_WORKERS)
    flat_out = _make_gather(total_b)(ids2, table)
    return flat_out.reshape(inputs.shape + (EMBED_DIM,))
